# Initial kernel scaffold; baseline (speedup 1.0000x reference)
import jax
import jax.numpy as jnp
from jax.experimental import pallas as pl


def kernel(inputs, embedding):
    raise NotImplementedError("write your pallas kernel here")


# speedup vs baseline: 9.0943x; 9.0943x over previous
import functools

import jax
import jax.numpy as jnp
from jax import lax
from jax.experimental import pallas as pl
from jax.experimental.pallas import tpu as pltpu
from jax.experimental.pallas import tpu_sc as plsc

_NUM_EMB = 8192
_DIM = 256
_N_TOK = 16384
_TOK_TILE = 256
_N_TILES = _N_TOK // _TOK_TILE
_COMMIT = 0.25


def _vq_body(x_ref, x2_ref, emb_ref, e2_ref,
             idx_ref, loss_ref, perp_ref,
             dsum_ref, counts_ref):
    i = pl.program_id(0)

    @pl.when(i == 0)
    def _init():
        dsum_ref[0, 0] = 0.0
        counts_ref[...] = jnp.zeros_like(counts_ref)

    x = x_ref[...]
    emb = emb_ref[...]
    m = lax.dot_general(x, emb, (((1,), (1,)), ((), ())),
                        preferred_element_type=jnp.float32)
    d = (x2_ref[...] + e2_ref[...]) - 2.0 * m
    min_d = jnp.min(d, axis=1, keepdims=True)
    iota = lax.broadcasted_iota(jnp.int32, (_TOK_TILE, _NUM_EMB), 1)
    idx = jnp.min(jnp.where(d == min_d, iota, jnp.int32(_NUM_EMB)), axis=1)
    idx_ref[0, 0, :] = idx

    dsum_ref[0, 0] += jnp.sum(min_d)
    onehot = (iota == idx[:, None]).astype(jnp.float32)
    counts_ref[0, :] += jnp.sum(onehot, axis=0)

    @pl.when(i == _N_TILES - 1)
    def _fin():
        loss = _COMMIT * (dsum_ref[0, 0] / (_N_TOK * _DIM))
        loss_ref[...] = loss.reshape(1, 1)
        avg = counts_ref[0, :] * (1.0 / _N_TOK)
        ent = -jnp.sum(avg * jnp.log(avg + 1e-10))
        perp_ref[...] = jnp.exp(ent).reshape(1, 1)


def _vq_argmin(flat, x2, embedding, e2):
    return pl.pallas_call(
        _vq_body,
        grid=(_N_TILES,),
        in_specs=[
            pl.BlockSpec((_TOK_TILE, _DIM), lambda i: (i, 0)),
            pl.BlockSpec((_TOK_TILE, 1), lambda i: (i, 0)),
            pl.BlockSpec((_NUM_EMB, _DIM), lambda i: (0, 0)),
            pl.BlockSpec((1, _NUM_EMB), lambda i: (0, 0)),
        ],
        out_specs=[
            pl.BlockSpec((1, 1, _TOK_TILE), lambda i: (i, 0, 0)),
            pl.BlockSpec((1, 1), lambda i: (0, 0)),
            pl.BlockSpec((1, 1), lambda i: (0, 0)),
        ],
        out_shape=[
            jax.ShapeDtypeStruct((_N_TILES, 1, _TOK_TILE), jnp.int32),
            jax.ShapeDtypeStruct((1, 1), jnp.float32),
            jax.ShapeDtypeStruct((1, 1), jnp.float32),
        ],
        scratch_shapes=[
            pltpu.SMEM((1, 1), jnp.float32),
            pltpu.VMEM((1, _NUM_EMB), jnp.float32),
        ],
    )(flat, x2, embedding, e2)


def _sc_gather(embedding, idx_flat):
    info = plsc.get_sparse_core_info()
    nw = info.num_cores * info.num_subcores
    b_per_w = _N_TOK // nw
    chunk = 128
    n_ch = b_per_w // chunk
    mesh = plsc.VectorSubcoreMesh(core_axis_name="c", subcore_axis_name="s")

    @functools.partial(
        pl.kernel, mesh=mesh,
        out_type=jax.ShapeDtypeStruct((_N_TOK, _DIM), jnp.float32),
        scratch_types=[
            pltpu.VMEM((b_per_w,), jnp.int32),
            pltpu.VMEM((chunk, _DIM), jnp.float32),
            pltpu.VMEM((chunk, _DIM), jnp.float32),
            pltpu.SemaphoreType.DMA,
            pltpu.SemaphoreType.DMA,
        ],
    )
    def gather_k(table_hbm, idx_hbm, out_hbm, idx_v, rows0, rows1, sem0, sem1):
        wid = lax.axis_index("s") * info.num_cores + lax.axis_index("c")
        base = wid * b_per_w
        pltpu.sync_copy(idx_hbm.at[pl.ds(base, b_per_w)], idx_v)
        bufs = (rows0, rows1)
        sems = (sem0, sem1)
        copies = []
        for c in range(n_ch):
            copies.append(pltpu.async_copy(
                table_hbm.at[idx_v.at[pl.ds(c * chunk, chunk)]],
                bufs[c % 2], sems[c % 2]))
            if c > 0:
                copies[c - 1].wait()
                pltpu.sync_copy(bufs[(c - 1) % 2],
                                out_hbm.at[pl.ds(base + (c - 1) * chunk, chunk)])
        copies[n_ch - 1].wait()
        pltpu.sync_copy(bufs[(n_ch - 1) % 2],
                        out_hbm.at[pl.ds(base + (n_ch - 1) * chunk, chunk)])

    return gather_k(embedding, idx_flat)


def kernel(inputs, embedding):
    x = jnp.transpose(inputs, (0, 2, 3, 1))
    input_shape = x.shape
    flat = x.reshape(-1, _DIM)
    x2 = jnp.sum(flat ** 2, axis=1, keepdims=True)
    e2 = jnp.sum(embedding ** 2, axis=1).reshape(1, _NUM_EMB)

    idx3, loss, perp = _vq_argmin(flat, x2, embedding, e2)
    idx_flat = idx3.reshape(-1)

    q = _sc_gather(embedding, idx_flat)

    quantized_out = jnp.transpose(q.reshape(input_shape), (0, 3, 1, 2))
    idx_out = idx_flat.reshape(input_shape[0], input_shape[1], input_shape[2])
    return (loss.reshape(()), quantized_out, perp.reshape(()), idx_out)
